# Initial kernel scaffold; baseline (speedup 1.0000x reference)
import jax
import jax.numpy as jnp
from jax.experimental import pallas as pl


def kernel(input, emb_weight):
    raise NotImplementedError("write your pallas kernel here")


# speedup vs baseline: 2.1234x; 2.1234x over previous
import jax
import jax.numpy as jnp
from jax import lax
from jax.experimental import pallas as pl
from jax.experimental.pallas import tpu as pltpu
from jax.experimental.pallas import tpu_sc as plsc

B = 4096
L = 100
E = 64
NC = 2
NS = 16
NW = NC * NS
RPW = B // NW


def _sc_body(inp_hbm, emb_hbm, out_hbm, emb_v, inp_v, out_v):
    c = lax.axis_index("c")
    s = lax.axis_index("s")
    wid = s * NC + c
    base = wid * RPW

    pltpu.sync_copy(emb_hbm, emb_v)
    pltpu.sync_copy(inp_hbm.at[pl.ds(base, RPW)], inp_v)

    def row_body(r, carry):
        row = inp_v[r]
        out_v[...] = row * emb_v[...]
        pltpu.sync_copy(out_v, out_hbm.at[base + r])
        return carry

    lax.fori_loop(0, RPW, row_body, 0)


def kernel(input, emb_weight):
    mesh = plsc.VectorSubcoreMesh(core_axis_name="c", subcore_axis_name="s")
    run = pl.kernel(
        _sc_body,
        mesh=mesh,
        compiler_params=pltpu.CompilerParams(use_tc_tiling_on_sc=False),
        out_type=jax.ShapeDtypeStruct((B, L, E), jnp.float32),
        scratch_types=[
            pltpu.VMEM((L, E), jnp.float32),
            pltpu.VMEM((RPW, L, 1), jnp.float32),
            pltpu.VMEM((L, E), jnp.float32),
        ],
    )
    out = run(input.reshape(B, L, 1), emb_weight)
    return out.reshape(B, 1, L * E)


# speedup vs baseline: 3.5606x; 1.6768x over previous
import jax
import jax.numpy as jnp
from jax import lax
from jax.experimental import pallas as pl
from jax.experimental.pallas import tpu as pltpu
from jax.experimental.pallas import tpu_sc as plsc

B = 4096
L = 100
E = 64
NC = 2
NS = 16
NW = NC * NS
RPW = B // NW


def _sc_body(inp_hbm, emb_hbm, out_hbm, emb_v, inp_v, out_v):
    c = lax.axis_index("c")
    s = lax.axis_index("s")
    wid = s * NC + c
    base = wid * RPW

    pltpu.sync_copy(emb_hbm, emb_v)
    pltpu.sync_copy(inp_hbm.at[pl.ds(base, RPW)], inp_v)

    def row_body(r, carry):
        row = inp_v[r]
        prod = row * emb_v[...]
        out_v[...] = prod.reshape(1, L * E)
        pltpu.sync_copy(out_v, out_hbm.at[base + r])
        return carry

    lax.fori_loop(0, RPW, row_body, 0)


def kernel(input, emb_weight):
    mesh = plsc.VectorSubcoreMesh(core_axis_name="c", subcore_axis_name="s")
    run = pl.kernel(
        _sc_body,
        mesh=mesh,
        compiler_params=pltpu.CompilerParams(use_tc_tiling_on_sc=False),
        out_type=jax.ShapeDtypeStruct((B, 1, L * E), jnp.float32),
        scratch_types=[
            pltpu.VMEM((L, E), jnp.float32),
            pltpu.VMEM((RPW, L, 1), jnp.float32),
            pltpu.VMEM((1, L * E), jnp.float32),
        ],
    )
    return run(input.reshape(B, L, 1), emb_weight)


# speedup vs baseline: 18.0087x; 5.0577x over previous
import jax
import jax.numpy as jnp
from jax import lax
from jax.experimental import pallas as pl
from jax.experimental.pallas import tpu as pltpu
from jax.experimental.pallas import tpu_sc as plsc

B = 4096
L = 100
E = 64
NC = 2
NS = 16
NW = NC * NS
RPW = B // NW
CH = 16
NCH = RPW // CH


def _sc_body(inp_hbm, emb_hbm, out_hbm, emb_v, inp_v, out_v):
    c = lax.axis_index("c")
    s = lax.axis_index("s")
    wid = s * NC + c
    base = wid * RPW

    pltpu.sync_copy(emb_hbm, emb_v)
    pltpu.sync_copy(inp_hbm.at[pl.ds(base, RPW)], inp_v)

    def chunk_body(ci, carry):
        def l_body(l, carry2):
            col = inp_v[pl.ds(ci * CH, CH), pl.ds(l, 1)]
            row = emb_v[pl.ds(l, 1)]
            prod = col * row
            out_v[:, :, pl.ds(l * E, E)] = prod.reshape(CH, 1, E)
            return carry2

        lax.fori_loop(0, L, l_body, 0)
        pltpu.sync_copy(out_v, out_hbm.at[pl.ds(base + ci * CH, CH)])
        return carry

    lax.fori_loop(0, NCH, chunk_body, 0)


def kernel(input, emb_weight):
    mesh = plsc.VectorSubcoreMesh(core_axis_name="c", subcore_axis_name="s")
    run = pl.kernel(
        _sc_body,
        mesh=mesh,
        compiler_params=pltpu.CompilerParams(use_tc_tiling_on_sc=False),
        out_type=jax.ShapeDtypeStruct((B, 1, L * E), jnp.float32),
        scratch_types=[
            pltpu.VMEM((L, E), jnp.float32),
            pltpu.VMEM((RPW, L), jnp.float32),
            pltpu.VMEM((CH, 1, L * E), jnp.float32),
        ],
    )
    return run(input, emb_weight)


# speedup vs baseline: 24.7131x; 1.3723x over previous
import jax
import jax.numpy as jnp
from jax import lax
from jax.experimental import pallas as pl
from jax.experimental.pallas import tpu as pltpu
from jax.experimental.pallas import tpu_sc as plsc

B = 4096
L = 100
E = 64
NC = 2
NS = 16
NW = NC * NS
RPW = B // NW
CH = 8
NCH = RPW // CH


def _sc_body(inp_hbm, emb_hbm, out_hbm, emb_v, inp_v, out_v, sem):
    c = lax.axis_index("c")
    s = lax.axis_index("s")
    wid = s * NC + c
    base = wid * RPW

    pltpu.sync_copy(emb_hbm, emb_v)
    pltpu.sync_copy(inp_hbm.at[pl.ds(base, RPW)], inp_v)

    def chunk_body(ci, carry):
        slot = lax.rem(ci, 2)

        @pl.when(ci >= 2)
        def _():
            pltpu.make_async_copy(
                out_v.at[slot], out_hbm.at[pl.ds(base, CH)], sem
            ).wait()

        def l_body(l, carry2):
            col = inp_v[pl.ds(ci * CH, CH), pl.ds(l, 1)]
            row = emb_v[pl.ds(l, 1)]
            prod = col * row
            out_v[slot, :, :, pl.ds(l * E, E)] = prod.reshape(CH, 1, E)
            return carry2

        lax.fori_loop(0, L, l_body, 0)
        pltpu.make_async_copy(
            out_v.at[slot], out_hbm.at[pl.ds(base + ci * CH, CH)], sem
        ).start()
        return carry

    lax.fori_loop(0, NCH, chunk_body, 0)
    pltpu.make_async_copy(out_v.at[0], out_hbm.at[pl.ds(base, CH)], sem).wait()
    pltpu.make_async_copy(out_v.at[1], out_hbm.at[pl.ds(base, CH)], sem).wait()


def kernel(input, emb_weight):
    mesh = plsc.VectorSubcoreMesh(core_axis_name="c", subcore_axis_name="s")
    run = pl.kernel(
        _sc_body,
        mesh=mesh,
        compiler_params=pltpu.CompilerParams(use_tc_tiling_on_sc=False),
        out_type=jax.ShapeDtypeStruct((B, 1, L * E), jnp.float32),
        scratch_types=[
            pltpu.VMEM((L, E), jnp.float32),
            pltpu.VMEM((RPW, L), jnp.float32),
            pltpu.VMEM((2, CH, 1, L * E), jnp.float32),
            pltpu.SemaphoreType.DMA,
        ],
    )
    return run(input, emb_weight)


# speedup vs baseline: 25.3201x; 1.0246x over previous
import jax
import jax.numpy as jnp
from jax import lax
from jax.experimental import pallas as pl
from jax.experimental.pallas import tpu as pltpu
from jax.experimental.pallas import tpu_sc as plsc

B = 4096
L = 100
LP = 128
E = 64
NC = 2
NS = 16
NW = NC * NS
RPW = B // NW
CH = 8
NCH = RPW // CH


def _sc_body(inp_hbm, emb_hbm, out_hbm, emb_v, inp_v, out_v, sem):
    c = lax.axis_index("c")
    s = lax.axis_index("s")
    wid = s * NC + c
    base = wid * RPW

    pltpu.sync_copy(emb_hbm, emb_v)
    pltpu.sync_copy(inp_hbm.at[pl.ds(base, RPW)], inp_v)

    def chunk_body(ci, carry):
        slot = lax.rem(ci, 2)

        @pl.when(ci >= 2)
        def _():
            pltpu.make_async_copy(
                out_v.at[slot], out_hbm.at[pl.ds(base, CH)], sem
            ).wait()

        def l_body(l, carry2):
            col = inp_v[pl.ds(ci * CH, CH), pl.ds(l, 1)]
            row = emb_v[pl.ds(l, 1)]
            prod = col * row
            out_v[slot, :, :, pl.ds(l * E, E)] = prod.reshape(CH, 1, E)
            return carry2

        lax.fori_loop(0, L, l_body, 0)
        pltpu.make_async_copy(
            out_v.at[slot], out_hbm.at[pl.ds(base + ci * CH, CH)], sem
        ).start()
        return carry

    lax.fori_loop(0, NCH, chunk_body, 0)
    pltpu.make_async_copy(out_v.at[0], out_hbm.at[pl.ds(base, CH)], sem).wait()
    pltpu.make_async_copy(out_v.at[1], out_hbm.at[pl.ds(base, CH)], sem).wait()


def kernel(input, emb_weight):
    inp_p = jnp.pad(input, ((0, 0), (0, LP - L)))
    mesh = plsc.VectorSubcoreMesh(core_axis_name="c", subcore_axis_name="s")
    run = pl.kernel(
        _sc_body,
        mesh=mesh,
        compiler_params=pltpu.CompilerParams(use_tc_tiling_on_sc=False),
        out_type=jax.ShapeDtypeStruct((B, 1, L * E), jnp.float32),
        scratch_types=[
            pltpu.VMEM((L, E), jnp.float32),
            pltpu.VMEM((RPW, LP), jnp.float32),
            pltpu.VMEM((2, CH, 1, L * E), jnp.float32),
            pltpu.SemaphoreType.DMA,
        ],
    )
    return run(inp_p, emb_weight)


# speedup vs baseline: 25.3446x; 1.0010x over previous
import jax
import jax.numpy as jnp
from jax import lax
from jax.experimental import pallas as pl
from jax.experimental.pallas import tpu as pltpu
from jax.experimental.pallas import tpu_sc as plsc

B = 4096
L = 100
LP = 128
E = 64
NC = 2
NS = 16
NW = NC * NS
RPW = B // NW
CH = 8
NCH = RPW // CH


def _sc_body(inp_hbm, emb_hbm, out_hbm, emb_v, inp_v, out_v, sem):
    c = lax.axis_index("c")
    s = lax.axis_index("s")
    wid = s * NC + c
    base = wid * RPW

    pltpu.sync_copy(emb_hbm, emb_v)
    pltpu.sync_copy(inp_hbm.at[pl.ds(base, RPW)], inp_v)

    def chunk_body(ci, carry):
        slot = lax.rem(ci, 2)

        @pl.when(ci >= 2)
        def _():
            pltpu.make_async_copy(
                out_v.at[slot], out_hbm.at[pl.ds(base, CH)], sem
            ).wait()

        def l_body(l, carry2):
            col = inp_v[pl.ds(ci * CH, CH), pl.ds(l, 1)]
            row = emb_v[pl.ds(l * E, E)]
            prod = col * row
            out_v[slot, :, :, pl.ds(l * E, E)] = prod.reshape(CH, 1, E)
            return carry2

        lax.fori_loop(0, L, l_body, 0)
        pltpu.make_async_copy(
            out_v.at[slot], out_hbm.at[pl.ds(base + ci * CH, CH)], sem
        ).start()
        return carry

    lax.fori_loop(0, NCH, chunk_body, 0)
    pltpu.make_async_copy(out_v.at[0], out_hbm.at[pl.ds(base, CH)], sem).wait()
    pltpu.make_async_copy(out_v.at[1], out_hbm.at[pl.ds(base, CH)], sem).wait()


def kernel(input, emb_weight):
    inp_p = jnp.concatenate(
        [input, jnp.zeros((B, LP - L), jnp.float32)], axis=1
    )
    emb_flat = emb_weight.reshape(L * E)
    mesh = plsc.VectorSubcoreMesh(core_axis_name="c", subcore_axis_name="s")
    run = pl.kernel(
        _sc_body,
        mesh=mesh,
        compiler_params=pltpu.CompilerParams(use_tc_tiling_on_sc=False),
        out_type=jax.ShapeDtypeStruct((B, 1, L * E), jnp.float32),
        scratch_types=[
            pltpu.VMEM((L * E,), jnp.float32),
            pltpu.VMEM((RPW, LP), jnp.float32),
            pltpu.VMEM((2, CH, 1, L * E), jnp.float32),
            pltpu.SemaphoreType.DMA,
        ],
    )
    return run(inp_p, emb_flat)
